# baseline (speedup 1.0000x reference)
import jax
import jax.numpy as jnp
from jax import lax
from jax.experimental import pallas as pl
from jax.experimental.pallas import tpu as pltpu
from jax.experimental.pallas import tpu_sc as plsc

NC = 2
NS = 16
NW = NC * NS
LANES = 16

B = 1024
L = 200
EMB = 64
V = 1000000
SHIFT = 491520
TROWS = 516096
TC_IB = 8192
N = B * L
PER_W = N // NW
CHUNK = 128
NCHUNK = PER_W // CHUNK
POSREP = 3200
COLV = EMB // LANES


def _pack_body(w1_ref, w2_ref, out_ref):
    r = jax.lax.broadcasted_iota(jnp.int32, (EMB, EMB), 0)
    c = jax.lax.broadcasted_iota(jnp.int32, (EMB, EMB), 1)
    eye = (r == c).astype(jnp.float32)
    dn = (((0,), (0,)), ((), ()))
    out_ref[:, 0:EMB] = jax.lax.dot_general(
        w1_ref[...], eye, dn, preferred_element_type=jnp.float32)
    out_ref[:, EMB:2 * EMB] = jax.lax.dot_general(
        w2_ref[...], eye, dn, preferred_element_type=jnp.float32)


def _pack_table(W):
    W_T = jnp.swapaxes(W, 0, 1)
    return pl.pallas_call(
        _pack_body,
        grid=(TROWS // TC_IB,),
        in_specs=[
            pl.BlockSpec((EMB, TC_IB), lambda i: (0, i)),
            pl.BlockSpec((EMB, TC_IB), lambda i: (0, i + SHIFT // TC_IB)),
        ],
        out_specs=pl.BlockSpec((TC_IB, 2 * EMB), lambda i: (i, 0)),
        out_shape=jax.ShapeDtypeStruct((TROWS, 2 * EMB), jnp.float32),
    )(W_T, W_T)


def _gather_body(jidx_hbm, hm_hbm, table_hbm, posrep_hbm, out_hbm,
                 jidx_v, hm_v, rows_v, out_v, gsem, wsem, psem, hsem):
    wid = lax.axis_index("s") * NC + lax.axis_index("c")
    base_row = wid * PER_W

    pltpu.sync_copy(jidx_hbm.at[pl.ds(base_row, PER_W)], jidx_v)

    def start_gather(k, b):
        pltpu.async_copy(
            hm_hbm.at[pl.ds((base_row + k * CHUNK) * LANES, CHUNK * LANES)],
            hm_v.at[b], hsem.at[b])
        return pltpu.async_copy(
            table_hbm.at[jidx_v.at[pl.ds(k * CHUNK, CHUNK)]],
            rows_v.at[b], gsem.at[b])

    def start_prefill(k, b):
        soff = lax.rem(k * CHUNK, POSREP)
        return pltpu.async_copy(posrep_hbm.at[pl.ds(soff, CHUNK)],
                                out_v.at[b], psem.at[b])

    def select_add(k, b):
        def row_body(r4, _):
            for u in range(4):
                r = r4 * 4 + u
                m = hm_v[b, pl.ds(r * LANES, LANES)] > 0
                for c in range(COLV):
                    lo = rows_v[b, r, pl.ds(c * LANES, LANES)]
                    hi = rows_v[b, r, pl.ds(EMB + c * LANES, LANES)]
                    v16 = jnp.where(m, hi, lo)
                    plsc.addupdate(
                        out_v.at[b, r, pl.ds(c * LANES, LANES)], v16)
            return 0

        lax.fori_loop(0, CHUNK // 4, row_body, 0)

    def chunk_step(k, b):
        @pl.when(k + 1 < NCHUNK)
        def _():
            start_gather(k + 1, 1 - b)

        pltpu.make_async_copy(
            table_hbm.at[jidx_v.at[pl.ds(0, CHUNK)]],
            rows_v.at[b], gsem.at[b]).wait()
        pltpu.make_async_copy(
            hm_hbm.at[pl.ds(0, CHUNK * LANES)], hm_v.at[b],
            hsem.at[b]).wait()
        pltpu.make_async_copy(
            posrep_hbm.at[pl.ds(0, CHUNK)], out_v.at[b], psem.at[b]).wait()
        select_add(k, b)
        pltpu.async_copy(
            out_v.at[b], out_hbm.at[pl.ds(base_row + k * CHUNK, CHUNK)],
            wsem.at[b])
        pltpu.make_async_copy(
            out_v.at[b], out_hbm.at[pl.ds(base_row, CHUNK)],
            wsem.at[b]).wait()

        @pl.when(k + 2 < NCHUNK)
        def _():
            start_prefill(k + 2, b)

    start_prefill(0, 0)
    start_prefill(1, 1)
    start_gather(0, 0)

    def g_body(g, _):
        chunk_step(g * 2, 0)
        chunk_step(g * 2 + 1, 1)
        return 0

    lax.fori_loop(0, NCHUNK // 2, g_body, 0)


@jax.jit
def _embed(x_flat, W, pos_rep):
    table = _pack_table(W)
    jidx = jnp.where(x_flat >= SHIFT, x_flat - SHIFT, x_flat)
    hm = jnp.repeat((x_flat >= SHIFT).astype(jnp.int32), LANES)
    mesh = plsc.VectorSubcoreMesh(core_axis_name="c", subcore_axis_name="s")
    k = pl.kernel(
        _gather_body,
        out_type=jax.ShapeDtypeStruct((N, EMB), jnp.float32),
        mesh=mesh,
        scratch_types=[
            pltpu.VMEM((PER_W,), jnp.int32),
            pltpu.VMEM((2, CHUNK * LANES), jnp.int32),
            pltpu.VMEM((2, CHUNK, 2 * EMB), jnp.float32),
            pltpu.VMEM((2, CHUNK, EMB), jnp.float32),
            pltpu.SemaphoreType.DMA((2,)),
            pltpu.SemaphoreType.DMA((2,)),
            pltpu.SemaphoreType.DMA((2,)),
            pltpu.SemaphoreType.DMA((2,)),
        ],
        compiler_params=pltpu.CompilerParams(use_tc_tiling_on_sc=True),
    )
    return k(jidx, hm, table, pos_rep)


def kernel(x, W, pos_emb):
    x_flat = x.reshape(-1).astype(jnp.int32)
    pos_rep = jnp.tile(pos_emb[:L], (POSREP // L, 1))
    out = _embed(x_flat, W, pos_rep)
    return out.reshape(x.shape[0], x.shape[1], EMB)


# speedup vs baseline: 1.2992x; 1.2992x over previous
import jax
import jax.numpy as jnp
from jax import lax
from jax.experimental import pallas as pl
from jax.experimental.pallas import tpu as pltpu
from jax.experimental.pallas import tpu_sc as plsc

NC = 2
NS = 16
NW = NC * NS
LANES = 16

B = 1024
L = 200
EMB = 64
V = 1000000
SHIFT = 491520
TROWS = 516096
TC_IB = 8192
N = B * L
PER_W = N // NW
CHUNK = 128
NCHUNK = PER_W // CHUNK
POSREP = 3200
COLV = EMB // LANES


def _pack_body(w1_ref, w2_ref, out_ref):
    r = jax.lax.broadcasted_iota(jnp.int32, (EMB, EMB), 0)
    c = jax.lax.broadcasted_iota(jnp.int32, (EMB, EMB), 1)
    eye = (r == c).astype(jnp.float32)
    dn = (((0,), (0,)), ((), ()))
    out_ref[:, 0:EMB] = jax.lax.dot_general(
        w1_ref[...], eye, dn, preferred_element_type=jnp.float32)
    out_ref[:, EMB:2 * EMB] = jax.lax.dot_general(
        w2_ref[...], eye, dn, preferred_element_type=jnp.float32)


def _pack_table(W):
    W_T = jnp.swapaxes(W, 0, 1)
    return pl.pallas_call(
        _pack_body,
        grid=(TROWS // TC_IB,),
        in_specs=[
            pl.BlockSpec((EMB, TC_IB), lambda i: (0, i)),
            pl.BlockSpec((EMB, TC_IB), lambda i: (0, i + SHIFT // TC_IB)),
        ],
        out_specs=pl.BlockSpec((TC_IB, 2 * EMB), lambda i: (i, 0)),
        out_shape=jax.ShapeDtypeStruct((TROWS, 2 * EMB), jnp.float32),
    )(W_T, W_T)


def _gather_body(jidx_hbm, off_hbm, table_hbm, posrep_hbm, out_hbm,
                 jidx_v, off_v, rows0, rows1, outv0, outv1,
                 gsem, wsem, psem, hsem):
    wid = lax.axis_index("s") * NC + lax.axis_index("c")
    base_row = wid * PER_W
    rows = (rows0, rows1)
    outs = (outv0, outv1)

    pltpu.sync_copy(jidx_hbm.at[pl.ds(base_row, PER_W)], jidx_v)

    def start_gather(k, b):
        pltpu.async_copy(
            off_hbm.at[pl.ds(base_row + k * CHUNK, CHUNK)],
            off_v.at[b, pl.ds(0, CHUNK)], hsem.at[b])
        return pltpu.async_copy(
            table_hbm.at[jidx_v.at[pl.ds(k * CHUNK, CHUNK)]],
            rows[b], gsem.at[b])

    def start_prefill(k, b):
        soff = lax.rem(k * CHUNK, POSREP)
        return pltpu.async_copy(posrep_hbm.at[pl.ds(soff, CHUNK)],
                                outs[b], psem.at[b])

    def select_add(k, b):
        def group_body(g, _):
            offs = off_v[b, pl.ds(g * LANES, LANES)]
            for u in range(LANES):
                r = g * LANES + u
                off = offs[u]
                for c in range(COLV):
                    v16 = rows[b][r, pl.ds(off + c * LANES, LANES)]
                    plsc.addupdate(
                        outs[b].at[r, pl.ds(c * LANES, LANES)], v16)
            return 0

        lax.fori_loop(0, CHUNK // LANES, group_body, 0)

    def chunk_step(k, b):
        @pl.when(k + 1 < NCHUNK)
        def _():
            start_gather(k + 1, 1 - b)

        pltpu.make_async_copy(
            table_hbm.at[jidx_v.at[pl.ds(0, CHUNK)]],
            rows[b], gsem.at[b]).wait()
        pltpu.make_async_copy(
            off_hbm.at[pl.ds(0, CHUNK)], off_v.at[b, pl.ds(0, CHUNK)],
            hsem.at[b]).wait()
        pltpu.make_async_copy(
            posrep_hbm.at[pl.ds(0, CHUNK)], outs[b], psem.at[b]).wait()
        select_add(k, b)
        pltpu.async_copy(
            outs[b], out_hbm.at[pl.ds(base_row + k * CHUNK, CHUNK)],
            wsem.at[b])
        pltpu.make_async_copy(
            outs[b], out_hbm.at[pl.ds(base_row, CHUNK)],
            wsem.at[b]).wait()

        @pl.when(k + 2 < NCHUNK)
        def _():
            start_prefill(k + 2, b)

    start_prefill(0, 0)
    start_prefill(1, 1)
    start_gather(0, 0)

    def g_body(g, _):
        chunk_step(g * 2, 0)
        chunk_step(g * 2 + 1, 1)
        return 0

    lax.fori_loop(0, NCHUNK // 2, g_body, 0)


@jax.jit
def _embed(x_flat, W, pos_rep):
    table = _pack_table(W)
    jidx = jnp.where(x_flat >= SHIFT, x_flat - SHIFT, x_flat)
    off = jnp.where(x_flat >= SHIFT, EMB, 0).astype(jnp.int32)
    mesh = plsc.VectorSubcoreMesh(core_axis_name="c", subcore_axis_name="s")
    k = pl.kernel(
        _gather_body,
        out_type=jax.ShapeDtypeStruct((N, EMB), jnp.float32),
        mesh=mesh,
        scratch_types=[
            pltpu.VMEM((PER_W,), jnp.int32),
            pltpu.VMEM((2, CHUNK + LANES), jnp.int32),
            pltpu.VMEM((CHUNK, 2 * EMB), jnp.float32),
            pltpu.VMEM((CHUNK, 2 * EMB), jnp.float32),
            pltpu.VMEM((CHUNK, EMB), jnp.float32),
            pltpu.VMEM((CHUNK, EMB), jnp.float32),
            pltpu.SemaphoreType.DMA((2,)),
            pltpu.SemaphoreType.DMA((2,)),
            pltpu.SemaphoreType.DMA((2,)),
            pltpu.SemaphoreType.DMA((2,)),
        ],
        compiler_params=pltpu.CompilerParams(use_tc_tiling_on_sc=True),
    )
    return k(jidx, off, table, pos_rep)


def kernel(x, W, pos_emb):
    x_flat = x.reshape(-1).astype(jnp.int32)
    pos_rep = jnp.tile(pos_emb[:L], (POSREP // L, 1))
    out = _embed(x_flat, W, pos_rep)
    return out.reshape(x.shape[0], x.shape[1], EMB)


# speedup vs baseline: 1.3241x; 1.0192x over previous
import jax
import jax.numpy as jnp
from jax import lax
from jax.experimental import pallas as pl
from jax.experimental.pallas import tpu as pltpu
from jax.experimental.pallas import tpu_sc as plsc

NC = 2
NS = 16
NW = NC * NS
LANES = 16

B = 1024
L = 200
EMB = 64
V = 1000000
SHIFT = 491520
TROWS = 524288
TC_IB = 16384
N = B * L
PER_W = N // NW
CHUNK = 128
NCHUNK = PER_W // CHUNK
POSREP = 3200
COLV = EMB // LANES


def _pack_body(w1_ref, w2_ref, out_ref):
    r = jax.lax.broadcasted_iota(jnp.int32, (EMB, EMB), 0)
    c = jax.lax.broadcasted_iota(jnp.int32, (EMB, EMB), 1)
    eye = (r == c).astype(jnp.float32)
    dn = (((0,), (0,)), ((), ()))
    out_ref[:, 0:EMB] = jax.lax.dot_general(
        w1_ref[...], eye, dn, preferred_element_type=jnp.float32)
    out_ref[:, EMB:2 * EMB] = jax.lax.dot_general(
        w2_ref[...], eye, dn, preferred_element_type=jnp.float32)


def _pack_table(W):
    W_T = jnp.swapaxes(W, 0, 1)
    return pl.pallas_call(
        _pack_body,
        grid=(TROWS // TC_IB,),
        in_specs=[
            pl.BlockSpec((EMB, TC_IB), lambda i: (0, i)),
            pl.BlockSpec((EMB, TC_IB), lambda i: (0, i + SHIFT // TC_IB)),
        ],
        out_specs=pl.BlockSpec((TC_IB, 2 * EMB), lambda i: (i, 0)),
        out_shape=jax.ShapeDtypeStruct((TROWS, 2 * EMB), jnp.float32),
    )(W_T, W_T)


def _gather_body(jidx_hbm, off_hbm, table_hbm, posrep_hbm, out_hbm,
                 jidx_v, off_v, rows0, rows1, outv0, outv1,
                 gsem, wsem, psem, hsem):
    wid = lax.axis_index("s") * NC + lax.axis_index("c")
    base_row = wid * PER_W
    rows = (rows0, rows1)
    outs = (outv0, outv1)

    pltpu.sync_copy(jidx_hbm.at[pl.ds(base_row, PER_W)], jidx_v)

    def start_gather(k, b):
        pltpu.async_copy(
            off_hbm.at[pl.ds(base_row + k * CHUNK, CHUNK)],
            off_v.at[b, pl.ds(0, CHUNK)], hsem.at[b])
        return pltpu.async_copy(
            table_hbm.at[jidx_v.at[pl.ds(k * CHUNK, CHUNK)]],
            rows[b], gsem.at[b])

    def start_prefill(k, b):
        soff = lax.rem(k * CHUNK, POSREP)
        return pltpu.async_copy(posrep_hbm.at[pl.ds(soff, CHUNK)],
                                outs[b], psem.at[b])

    def select_add(k, b):
        def group_body(g, _):
            offs = off_v[b, pl.ds(g * LANES, LANES)]
            for u in range(LANES):
                r = g * LANES + u
                off = offs[u]
                for c in range(COLV):
                    v16 = rows[b][r, pl.ds(off + c * LANES, LANES)]
                    plsc.addupdate(
                        outs[b].at[r, pl.ds(c * LANES, LANES)], v16)
            return 0

        lax.fori_loop(0, CHUNK // LANES, group_body, 0)

    def chunk_step(k, b):
        @pl.when(k + 1 < NCHUNK)
        def _():
            start_gather(k + 1, 1 - b)

        pltpu.make_async_copy(
            table_hbm.at[jidx_v.at[pl.ds(0, CHUNK)]],
            rows[b], gsem.at[b]).wait()
        pltpu.make_async_copy(
            off_hbm.at[pl.ds(0, CHUNK)], off_v.at[b, pl.ds(0, CHUNK)],
            hsem.at[b]).wait()
        pltpu.make_async_copy(
            posrep_hbm.at[pl.ds(0, CHUNK)], outs[b], psem.at[b]).wait()
        select_add(k, b)
        pltpu.async_copy(
            outs[b], out_hbm.at[pl.ds(base_row + k * CHUNK, CHUNK)],
            wsem.at[b])
        pltpu.make_async_copy(
            outs[b], out_hbm.at[pl.ds(base_row, CHUNK)],
            wsem.at[b]).wait()

        @pl.when(k + 2 < NCHUNK)
        def _():
            start_prefill(k + 2, b)

    start_prefill(0, 0)
    start_prefill(1, 1)
    start_gather(0, 0)

    def g_body(g, _):
        chunk_step(g * 2, 0)
        chunk_step(g * 2 + 1, 1)
        return 0

    lax.fori_loop(0, NCHUNK // 2, g_body, 0)


@jax.jit
def _embed(x_flat, W, pos_rep):
    table = _pack_table(W)
    jidx = jnp.where(x_flat >= SHIFT, x_flat - SHIFT, x_flat)
    off = jnp.where(x_flat >= SHIFT, EMB, 0).astype(jnp.int32)
    mesh = plsc.VectorSubcoreMesh(core_axis_name="c", subcore_axis_name="s")
    k = pl.kernel(
        _gather_body,
        out_type=jax.ShapeDtypeStruct((N, EMB), jnp.float32),
        mesh=mesh,
        scratch_types=[
            pltpu.VMEM((PER_W,), jnp.int32),
            pltpu.VMEM((2, CHUNK + LANES), jnp.int32),
            pltpu.VMEM((CHUNK, 2 * EMB), jnp.float32),
            pltpu.VMEM((CHUNK, 2 * EMB), jnp.float32),
            pltpu.VMEM((CHUNK, EMB), jnp.float32),
            pltpu.VMEM((CHUNK, EMB), jnp.float32),
            pltpu.SemaphoreType.DMA((2,)),
            pltpu.SemaphoreType.DMA((2,)),
            pltpu.SemaphoreType.DMA((2,)),
            pltpu.SemaphoreType.DMA((2,)),
        ],
        compiler_params=pltpu.CompilerParams(use_tc_tiling_on_sc=True),
    )
    return k(jidx, off, table, pos_rep)


def kernel(x, W, pos_emb):
    x_flat = x.reshape(-1).astype(jnp.int32)
    pos_rep = jnp.tile(pos_emb[:L], (POSREP // L, 1))
    out = _embed(x_flat, W, pos_rep)
    return out.reshape(x.shape[0], x.shape[1], EMB)
